# baseline (speedup 1.0000x reference)
import functools

import jax
import jax.numpy as jnp
from jax import lax
from jax.experimental import pallas as pl
from jax.experimental.pallas import tpu as pltpu
from jax.experimental.pallas import tpu_sc as plsc

NU = 8
H = 128
T = 10
BUFN = 50000
D = 512
PER = D // NU
B = 4096
TP = 128

CHUNKS = B // 16


def _tc_body(data_ref, bw_ref, bb_ref, lw_ref, lb_ref, y_ref, w_sc, c_sc):
    @pl.when(pl.program_id(0) == 0)
    def _():
        c = jnp.pad(lb_ref[...], ((0, 0), (0, TP - T)))
        for u in range(NU):
            lw_u = jnp.pad(lw_ref[pl.ds(u * H, H), :], ((0, 0), (0, TP - T)))
            w_sc[pl.ds(u * PER, PER), :] = jnp.dot(
                bw_ref[u], lw_u, preferred_element_type=jnp.float32)
            c = c + jnp.dot(bb_ref[pl.ds(u, 1), :], lw_u,
                            preferred_element_type=jnp.float32)
        c_sc[...] = c

    y_ref[...] = jnp.dot(data_ref[...], w_sc[...],
                         preferred_element_type=jnp.float32) + c_sc[...]


NROW = B // 128
ROWS_PER_SUB = NROW // 16
OW = TP


def _sc_perm_body(ids_hbm, perm_hbm, ids_v, table_v, perm_v):
    cid = lax.axis_index("c")
    sid = lax.axis_index("s")

    @pl.when(jnp.logical_and(cid == 0, sid == 0))
    def _():
        pltpu.sync_copy(ids_hbm, ids_v)

        def scatter_step(i, carry):
            for u in range(8):
                base = (i * 8 + u) * 16
                idx = ids_v[pl.ds(base, 16)]
                vals = base + jnp.arange(16, dtype=jnp.int32)
                plsc.store_scatter(table_v, [idx], vals)
            return carry

        lax.fori_loop(0, CHUNKS // 8, scatter_step, 0)

        def fix_pass(_):
            def fp(j, acc):
                row = perm_v.at[j]
                for k in range(8):
                    base = j * 128 + k * 16
                    idx = ids_v[pl.ds(base, 16)]
                    vals = base + jnp.arange(16, dtype=jnp.int32)
                    g = plsc.load_gather(table_v, [idx])
                    m = g < vals
                    plsc.store_scatter(table_v, [idx], vals, mask=m)
                    row[pl.ds(k * 16, 16)] = jnp.where(m, vals, g)
                    acc = acc | m.astype(jnp.int32)
                return acc

            acc = lax.fori_loop(0, NROW, fp, jnp.zeros((16,), jnp.int32))
            return jnp.max(acc) > 0

        lax.while_loop(lambda c: c, fix_pass, fix_pass(True))
        pltpu.sync_copy(perm_v, perm_hbm)


def _sc_gather_body(y_hbm, perm_hbm, out_hbm, idx1_v, rows_v, sem):
    cid = lax.axis_index("c")
    sid = lax.axis_index("s")

    del cid
    for t in range(ROWS_PER_SUB):
        j = sid * ROWS_PER_SUB + t
        pltpu.sync_copy(perm_hbm.at[j], idx1_v.at[t])
        pltpu.async_copy(y_hbm.at[idx1_v.at[t]], rows_v.at[t], sem).wait()
        pltpu.sync_copy(rows_v.at[t], out_hbm.at[pl.ds(j * 128, 128)])


@functools.cache
def _sc_perm_kernel():
    return pl.kernel(
        _sc_perm_body,
        out_type=jax.ShapeDtypeStruct((NROW, 128), jnp.int32),
        mesh=plsc.VectorSubcoreMesh(core_axis_name="c", subcore_axis_name="s",
                                    num_cores=1),
        scratch_types=[
            pltpu.VMEM((B,), jnp.int32),
            pltpu.VMEM((BUFN,), jnp.int32),
            pltpu.VMEM((NROW, 128), jnp.int32),
        ],
        compiler_params=pltpu.CompilerParams(needs_layout_passes=False),
    )


@functools.cache
def _sc_gather_kernel():
    return pl.kernel(
        _sc_gather_body,
        out_type=jax.ShapeDtypeStruct((B, OW), jnp.float32),
        mesh=plsc.VectorSubcoreMesh(core_axis_name="c", subcore_axis_name="s",
                                    num_cores=1),
        scratch_types=[
            pltpu.VMEM((ROWS_PER_SUB, 128), jnp.int32),
            pltpu.VMEM((ROWS_PER_SUB, 128, TP), jnp.float32),
            pltpu.SemaphoreType.DMA,
        ],
        compiler_params=pltpu.CompilerParams(needs_layout_passes=False),
    )


GB = 8


def kernel(data, ids, target, block_w, block_b, lin_w, lin_b, buffer):
    del target, buffer
    perm = _sc_perm_kernel()(ids.astype(jnp.int32))
    y = pl.pallas_call(
        _tc_body,
        grid=(GB,),
        in_specs=[
            pl.BlockSpec((B // GB, D), lambda i: (i, 0)),
            pl.BlockSpec((NU, PER, H), lambda i: (0, 0, 0)),
            pl.BlockSpec((NU, H), lambda i: (0, 0)),
            pl.BlockSpec((NU * H, T), lambda i: (0, 0)),
            pl.BlockSpec((1, T), lambda i: (0, 0)),
        ],
        out_specs=pl.BlockSpec((B // GB, TP), lambda i: (i, 0)),
        scratch_shapes=[
            pltpu.VMEM((D, TP), jnp.float32),
            pltpu.VMEM((1, TP), jnp.float32),
        ],
        out_shape=jax.ShapeDtypeStruct((B, TP), jnp.float32),
    )(data, block_w, block_b, lin_w, lin_b.reshape(1, T))
    out = _sc_gather_kernel()(y, perm)
    return out[:, :T]


# speedup vs baseline: 1.0042x; 1.0042x over previous
import functools

import jax
import jax.numpy as jnp
from jax import lax
from jax.experimental import pallas as pl
from jax.experimental.pallas import tpu as pltpu
from jax.experimental.pallas import tpu_sc as plsc

NU = 8
H = 128
T = 10
BUFN = 50000
D = 512
PER = D // NU
B = 4096
TP = 128

CHUNKS = B // 16


def _tc_body(data_ref, bw_ref, bb_ref, lw_ref, lb_ref, y_ref):
    pieces = []
    c = jnp.pad(lb_ref[...], ((0, 0), (0, TP - T)))
    for u in range(NU):
        lw_u = jnp.pad(lw_ref[pl.ds(u * H, H), :], ((0, 0), (0, TP - T)))
        pieces.append(jnp.dot(bw_ref[u], lw_u,
                              preferred_element_type=jnp.float32))
        c = c + jnp.dot(bb_ref[pl.ds(u, 1), :], lw_u,
                        preferred_element_type=jnp.float32)
    w_eff = jnp.concatenate(pieces, axis=0)
    y_ref[...] = jnp.dot(data_ref[...], w_eff,
                         preferred_element_type=jnp.float32) + c


NROW = B // 128
ROWS_PER_SUB = NROW // 16
OW = TP


def _sc_perm_body(ids_hbm, perm_hbm, ids_v, table_v, perm_v):
    cid = lax.axis_index("c")
    sid = lax.axis_index("s")

    @pl.when(jnp.logical_and(cid == 0, sid == 0))
    def _():
        pltpu.sync_copy(ids_hbm, ids_v)

        def scatter_step(i, carry):
            for u in range(8):
                base = (i * 8 + u) * 16
                idx = ids_v[pl.ds(base, 16)]
                vals = base + jnp.arange(16, dtype=jnp.int32)
                plsc.store_scatter(table_v, [idx], vals)
            return carry

        lax.fori_loop(0, CHUNKS // 8, scatter_step, 0)

        def fix_pass(_):
            def fp(j, acc):
                row = perm_v.at[j]
                for k in range(8):
                    base = j * 128 + k * 16
                    idx = ids_v[pl.ds(base, 16)]
                    vals = base + jnp.arange(16, dtype=jnp.int32)
                    g = plsc.load_gather(table_v, [idx])
                    m = g < vals
                    plsc.store_scatter(table_v, [idx], vals, mask=m)
                    row[pl.ds(k * 16, 16)] = jnp.where(m, vals, g)
                    acc = acc | m.astype(jnp.int32)
                return acc

            acc = lax.fori_loop(0, NROW, fp, jnp.zeros((16,), jnp.int32))
            return jnp.max(acc) > 0

        lax.while_loop(lambda c: c, fix_pass, fix_pass(True))
        pltpu.sync_copy(perm_v, perm_hbm)


def _sc_gather_body(y_hbm, perm_hbm, out_hbm, idx1_v, rows_v, sem):
    cid = lax.axis_index("c")
    sid = lax.axis_index("s")

    j = sid * 2 + cid
    pltpu.sync_copy(perm_hbm.at[j], idx1_v)
    pltpu.async_copy(y_hbm.at[idx1_v], rows_v, sem).wait()
    pltpu.sync_copy(rows_v, out_hbm.at[pl.ds(j * 128, 128)])


@functools.cache
def _sc_perm_kernel():
    return pl.kernel(
        _sc_perm_body,
        out_type=jax.ShapeDtypeStruct((NROW, 128), jnp.int32),
        mesh=plsc.VectorSubcoreMesh(core_axis_name="c", subcore_axis_name="s"),
        scratch_types=[
            pltpu.VMEM((B,), jnp.int32),
            pltpu.VMEM((BUFN,), jnp.int32),
            pltpu.VMEM((NROW, 128), jnp.int32),
        ],
        compiler_params=pltpu.CompilerParams(needs_layout_passes=False),
    )


@functools.cache
def _sc_gather_kernel():
    return pl.kernel(
        _sc_gather_body,
        out_type=jax.ShapeDtypeStruct((B, OW), jnp.float32),
        mesh=plsc.VectorSubcoreMesh(core_axis_name="c", subcore_axis_name="s"),
        scratch_types=[
            pltpu.VMEM((128,), jnp.int32),
            pltpu.VMEM((128, TP), jnp.float32),
            pltpu.SemaphoreType.DMA,
        ],
        compiler_params=pltpu.CompilerParams(needs_layout_passes=False),
    )


GB = 8


def kernel(data, ids, target, block_w, block_b, lin_w, lin_b, buffer):
    del target, buffer
    perm = _sc_perm_kernel()(ids.astype(jnp.int32))
    y = pl.pallas_call(
        _tc_body,
        out_shape=jax.ShapeDtypeStruct((B, TP), jnp.float32),
    )(data, block_w, block_b, lin_w, lin_b.reshape(1, T))
    out = _sc_gather_kernel()(y, perm)
    return out[:, :T]


# speedup vs baseline: 1.0257x; 1.0214x over previous
import functools

import jax
import jax.numpy as jnp
from jax import lax
from jax.experimental import pallas as pl
from jax.experimental.pallas import tpu as pltpu
from jax.experimental.pallas import tpu_sc as plsc

NU = 8
H = 128
T = 10
BUFN = 50000
D = 512
PER = D // NU
B = 4096
TP = 128

CHUNKS = B // 16


def _tc_body(data_ref, bw_ref, bb_ref, lw_ref, lb_ref, y_ref, w_sc, c_sc):
    @pl.when(pl.program_id(0) == 0)
    def _():
        c = jnp.pad(lb_ref[...], ((0, 0), (0, TP - T)))
        for u in range(NU):
            lw_u = jnp.pad(lw_ref[pl.ds(u * H, H), :], ((0, 0), (0, TP - T)))
            w_sc[pl.ds(u * PER, PER), :] = jnp.dot(
                bw_ref[u], lw_u, preferred_element_type=jnp.float32)
            c = c + jnp.dot(bb_ref[pl.ds(u, 1), :], lw_u,
                            preferred_element_type=jnp.float32)
        c_sc[...] = c

    y_ref[...] = jnp.dot(data_ref[...], w_sc[...],
                         preferred_element_type=jnp.float32) + c_sc[...]


NROW = B // 128
ROWS_PER_SUB = NROW // 16
OW = TP


def _sc_perm_body(ids_hbm, perm_hbm, ids_v, table_v, perm_v):
    cid = lax.axis_index("c")
    sid = lax.axis_index("s")

    @pl.when(jnp.logical_and(cid == 0, sid == 0))
    def _():
        pltpu.sync_copy(ids_hbm, ids_v)

        def scatter_step(i, carry):
            for u in range(8):
                base = (i * 8 + u) * 16
                idx = ids_v[pl.ds(base, 16)]
                vals = base + jnp.arange(16, dtype=jnp.int32)
                plsc.store_scatter(table_v, [idx], vals)
            return carry

        lax.fori_loop(0, CHUNKS // 8, scatter_step, 0)

        def fix_pass(_):
            def fp(j, acc):
                row = perm_v.at[j]
                for k in range(8):
                    base = j * 128 + k * 16
                    idx = ids_v[pl.ds(base, 16)]
                    vals = base + jnp.arange(16, dtype=jnp.int32)
                    g = plsc.load_gather(table_v, [idx])
                    m = g < vals
                    plsc.store_scatter(table_v, [idx], vals, mask=m)
                    row[pl.ds(k * 16, 16)] = jnp.where(m, vals, g)
                    acc = acc | m.astype(jnp.int32)
                return acc

            acc = lax.fori_loop(0, NROW, fp, jnp.zeros((16,), jnp.int32))
            return jnp.max(acc) > 0

        lax.while_loop(lambda c: c, fix_pass, fix_pass(True))
        pltpu.sync_copy(perm_v, perm_hbm)


def _sc_gather_body(y_hbm, perm_hbm, out_hbm, idx1_v, rows_v, sem):
    cid = lax.axis_index("c")
    sid = lax.axis_index("s")

    j = sid * 2 + cid
    pltpu.sync_copy(perm_hbm.at[j], idx1_v)
    pltpu.async_copy(y_hbm.at[idx1_v], rows_v, sem).wait()
    pltpu.sync_copy(rows_v, out_hbm.at[pl.ds(j * 128, 128)])


@functools.cache
def _sc_perm_kernel():
    return pl.kernel(
        _sc_perm_body,
        out_type=jax.ShapeDtypeStruct((NROW, 128), jnp.int32),
        mesh=plsc.VectorSubcoreMesh(core_axis_name="c", subcore_axis_name="s"),
        scratch_types=[
            pltpu.VMEM((B,), jnp.int32),
            pltpu.VMEM((BUFN,), jnp.int32),
            pltpu.VMEM((NROW, 128), jnp.int32),
        ],
        compiler_params=pltpu.CompilerParams(needs_layout_passes=False),
    )


@functools.cache
def _sc_gather_kernel():
    return pl.kernel(
        _sc_gather_body,
        out_type=jax.ShapeDtypeStruct((B, OW), jnp.float32),
        mesh=plsc.VectorSubcoreMesh(core_axis_name="c", subcore_axis_name="s"),
        scratch_types=[
            pltpu.VMEM((128,), jnp.int32),
            pltpu.VMEM((128, TP), jnp.float32),
            pltpu.SemaphoreType.DMA,
        ],
        compiler_params=pltpu.CompilerParams(needs_layout_passes=False),
    )


GB = 8


def kernel(data, ids, target, block_w, block_b, lin_w, lin_b, buffer):
    del target, buffer
    perm = _sc_perm_kernel()(ids.astype(jnp.int32))
    y = pl.pallas_call(
        _tc_body,
        grid=(GB,),
        in_specs=[
            pl.BlockSpec((B // GB, D), lambda i: (i, 0)),
            pl.BlockSpec((NU, PER, H), lambda i: (0, 0, 0)),
            pl.BlockSpec((NU, H), lambda i: (0, 0)),
            pl.BlockSpec((NU * H, T), lambda i: (0, 0)),
            pl.BlockSpec((1, T), lambda i: (0, 0)),
        ],
        out_specs=pl.BlockSpec((B // GB, TP), lambda i: (i, 0)),
        scratch_shapes=[
            pltpu.VMEM((D, TP), jnp.float32),
            pltpu.VMEM((1, TP), jnp.float32),
        ],
        out_shape=jax.ShapeDtypeStruct((B, TP), jnp.float32),
    )(data, block_w, block_b, lin_w, lin_b.reshape(1, T))
    out = _sc_gather_kernel()(y, perm)
    return out[:, :T]
